# Initial kernel scaffold; baseline (speedup 1.0000x reference)
import jax
import jax.numpy as jnp
from jax.experimental import pallas as pl


def kernel(task_repr, task_edge, W, b):
    raise NotImplementedError("write your pallas kernel here")


# speedup vs baseline: 15.5623x; 15.5623x over previous
import functools

import jax
import jax.numpy as jnp
from jax import lax
from jax.experimental import pallas as pl
from jax.experimental.pallas import tpu as pltpu
from jax.experimental.pallas import tpu_sc as plsc

N_NODES = 10000
N_EDGES = 160000
D_FEAT = 256

_ROWS_PER_BLK = 1000

_info = plsc.get_sparse_core_info()
_NC, _NS, _L = _info.num_cores, _info.num_subcores, _info.num_lanes
_NW = _NC * _NS
_EPW = N_EDGES // _NW
_FULL_ITERS = _EPW // _L
_TAIL = _EPW - _FULL_ITERS * _L
_EPW_PAD = (_FULL_ITERS + (1 if _TAIL else 0)) * _L


def _matvec_body(x_ref, w_ref, db_ref, o_ref):
    o_ref[...] = (
        jnp.dot(x_ref[...], w_ref[...], preferred_element_type=jnp.float32)
        + db_ref[0, 0]
    )


def _node_scores(task_repr, wv, db):
    return pl.pallas_call(
        _matvec_body,
        grid=(N_NODES // _ROWS_PER_BLK,),
        in_specs=[
            pl.BlockSpec((_ROWS_PER_BLK, D_FEAT), lambda i: (i, 0)),
            pl.BlockSpec((D_FEAT, 1), lambda i: (0, 0)),
            pl.BlockSpec((1, 1), lambda i: (0, 0)),
        ],
        out_specs=pl.BlockSpec((_ROWS_PER_BLK, 1), lambda i: (i, 0)),
        out_shape=jax.ShapeDtypeStruct((N_NODES, 1), jnp.float32),
    )(task_repr, wv, db)


_sc_mesh = plsc.VectorSubcoreMesh(core_axis_name="c", subcore_axis_name="s")


@functools.partial(
    pl.kernel,
    out_type=(
        jax.ShapeDtypeStruct((N_EDGES,), jnp.float32),
        jax.ShapeDtypeStruct((N_EDGES,), jnp.float32),
    ),
    mesh=_sc_mesh,
    compiler_params=pltpu.CompilerParams(needs_layout_passes=False),
    scratch_types=[
        pltpu.VMEM((N_NODES,), jnp.float32),
        pltpu.VMEM((_EPW_PAD,), jnp.int32),
        pltpu.VMEM((_EPW_PAD,), jnp.int32),
        pltpu.VMEM((_EPW_PAD,), jnp.float32),
        pltpu.VMEM((_EPW_PAD,), jnp.float32),
    ],
)
def _edge_softmax(d_hbm, row_hbm, col_hbm, ac_hbm, at_hbm,
                  d_v, row_v, col_v, ac_v, at_v):
    wid = lax.axis_index("s") * _NC + lax.axis_index("c")
    base = wid * _EPW
    pltpu.sync_copy(d_hbm, d_v)
    pltpu.sync_copy(row_hbm.at[pl.ds(base, _EPW)], row_v.at[pl.ds(0, _EPW)])
    pltpu.sync_copy(col_hbm.at[pl.ds(base, _EPW)], col_v.at[pl.ds(0, _EPW)])

    if _TAIL:
        lane = lax.iota(jnp.int32, _L)
        keep = lane < _TAIL
        toff = _FULL_ITERS * _L
        row_v[pl.ds(toff, _L)] = jnp.where(keep, row_v[pl.ds(toff, _L)], 0)
        col_v[pl.ds(toff, _L)] = jnp.where(keep, col_v[pl.ds(toff, _L)], 0)

    def body(i, carry):
        off = i * _L
        r = row_v[pl.ds(off, _L)]
        c = col_v[pl.ds(off, _L)]
        s = plsc.load_gather(d_v, [r]) + plsc.load_gather(d_v, [c])
        ac_v[pl.ds(off, _L)] = 1.0 / (1.0 + jnp.exp(-s))
        at_v[pl.ds(off, _L)] = 1.0 / (1.0 + jnp.exp(s))
        return carry

    lax.fori_loop(0, _FULL_ITERS + (1 if _TAIL else 0), body, 0)

    pltpu.sync_copy(ac_v.at[pl.ds(0, _EPW)], ac_hbm.at[pl.ds(base, _EPW)])
    pltpu.sync_copy(at_v.at[pl.ds(0, _EPW)], at_hbm.at[pl.ds(base, _EPW)])


def kernel(task_repr, task_edge, W, b):
    wv = (W[:, 0] - W[:, 1]).reshape(D_FEAT, 1)
    db = ((b[0] - b[1]) * 0.5).reshape(1, 1)
    d = _node_scores(task_repr, wv, db).reshape(N_NODES)
    row = task_edge[0].astype(jnp.int32)
    col = task_edge[1].astype(jnp.int32)
    a_causual, a_trivial = _edge_softmax(d, row, col)
    return (a_causual, a_trivial)


# speedup vs baseline: 16.4759x; 1.0587x over previous
import functools

import jax
import jax.numpy as jnp
from jax import lax
from jax.experimental import pallas as pl
from jax.experimental.pallas import tpu as pltpu
from jax.experimental.pallas import tpu_sc as plsc

N_NODES = 10000
N_EDGES = 160000
D_FEAT = 256

_ROWS_PER_BLK = 1000

_info = plsc.get_sparse_core_info()
_NC, _NS, _L = _info.num_cores, _info.num_subcores, _info.num_lanes
_NW = _NC * _NS
_EPW = N_EDGES // _NW
_UNROLL = 4
_FULL_ITERS = _EPW // (_L * _UNROLL)
_EPW_PAD = (_EPW + _L - 1) // _L * _L


def _matvec_body(x_ref, w_ref, b_ref, o_ref):
    wv = w_ref[:, 0:1] - w_ref[:, 1:2]
    db = (b_ref[0, 0] - b_ref[0, 1]) * 0.5
    o_ref[...] = (
        jnp.dot(x_ref[...], wv, preferred_element_type=jnp.float32) + db
    )


def _node_scores(task_repr, W, b2):
    return pl.pallas_call(
        _matvec_body,
        grid=(N_NODES // _ROWS_PER_BLK,),
        in_specs=[
            pl.BlockSpec((_ROWS_PER_BLK, D_FEAT), lambda i: (i, 0)),
            pl.BlockSpec((D_FEAT, 2), lambda i: (0, 0)),
            pl.BlockSpec((1, 2), lambda i: (0, 0)),
        ],
        out_specs=pl.BlockSpec((_ROWS_PER_BLK, 1), lambda i: (i, 0)),
        out_shape=jax.ShapeDtypeStruct((N_NODES, 1), jnp.float32),
    )(task_repr, W, b2)


_sc_mesh = plsc.VectorSubcoreMesh(core_axis_name="c", subcore_axis_name="s")


@functools.partial(
    pl.kernel,
    out_type=(
        jax.ShapeDtypeStruct((N_EDGES,), jnp.float32),
        jax.ShapeDtypeStruct((N_EDGES,), jnp.float32),
    ),
    mesh=_sc_mesh,
    compiler_params=pltpu.CompilerParams(needs_layout_passes=False),
    scratch_types=[
        pltpu.VMEM((N_NODES,), jnp.float32),
        pltpu.VMEM((_EPW_PAD,), jnp.int32),
        pltpu.VMEM((_EPW_PAD,), jnp.int32),
        pltpu.VMEM((_EPW_PAD,), jnp.float32),
        pltpu.VMEM((_EPW_PAD,), jnp.float32),
        pltpu.SemaphoreType.DMA,
        pltpu.SemaphoreType.DMA,
        pltpu.SemaphoreType.DMA,
    ],
)
def _edge_softmax(d_hbm, row_hbm, col_hbm, ac_hbm, at_hbm,
                  d_v, row_v, col_v, ac_v, at_v, sem0, sem1, sem2):
    wid = lax.axis_index("s") * _NC + lax.axis_index("c")
    base = wid * _EPW
    cp_d = pltpu.async_copy(d_hbm, d_v, sem0)
    cp_r = pltpu.async_copy(
        row_hbm.at[pl.ds(base, _EPW)], row_v.at[pl.ds(0, _EPW)], sem1)
    cp_c = pltpu.async_copy(
        col_hbm.at[pl.ds(base, _EPW)], col_v.at[pl.ds(0, _EPW)], sem2)
    cp_d.wait()
    cp_r.wait()
    cp_c.wait()

    def lanes(off):
        r = row_v[pl.ds(off, _L)]
        c = col_v[pl.ds(off, _L)]
        s = plsc.load_gather(d_v, [r]) + plsc.load_gather(d_v, [c])
        ac = 1.0 / (1.0 + jnp.exp(-s))
        ac_v[pl.ds(off, _L)] = ac
        at_v[pl.ds(off, _L)] = 1.0 - ac

    def body(i, carry):
        for j in range(_UNROLL):
            lanes(i * (_L * _UNROLL) + j * _L)
        return carry

    lax.fori_loop(0, _FULL_ITERS, body, 0)

    lane = lax.iota(jnp.int32, _L)
    for off in range(_FULL_ITERS * _L * _UNROLL, _EPW_PAD, _L):
        n_valid = _EPW - off
        if n_valid >= _L:
            lanes(off)
        else:
            keep = lane < n_valid
            r = jnp.where(keep, row_v[pl.ds(off, _L)], 0)
            c = jnp.where(keep, col_v[pl.ds(off, _L)], 0)
            s = plsc.load_gather(d_v, [r]) + plsc.load_gather(d_v, [c])
            ac = 1.0 / (1.0 + jnp.exp(-s))
            ac_v[pl.ds(off, _L)] = ac
            at_v[pl.ds(off, _L)] = 1.0 - ac

    cp_ac = pltpu.async_copy(
        ac_v.at[pl.ds(0, _EPW)], ac_hbm.at[pl.ds(base, _EPW)], sem0)
    cp_at = pltpu.async_copy(
        at_v.at[pl.ds(0, _EPW)], at_hbm.at[pl.ds(base, _EPW)], sem1)
    cp_ac.wait()
    cp_at.wait()


def kernel(task_repr, task_edge, W, b):
    d = _node_scores(task_repr, W, b.reshape(1, 2)).reshape(N_NODES)
    row = task_edge[0].astype(jnp.int32)
    col = task_edge[1].astype(jnp.int32)
    a_causual, a_trivial = _edge_softmax(d, row, col)
    return (a_causual, a_trivial)


# speedup vs baseline: 20.5973x; 1.2502x over previous
import functools

import jax
import jax.numpy as jnp
from jax import lax
from jax.experimental import pallas as pl
from jax.experimental.pallas import tpu as pltpu
from jax.experimental.pallas import tpu_sc as plsc

N_NODES = 10000
N_EDGES = 160000
D_FEAT = 256

_ROWS_PER_BLK = 2048

_info = plsc.get_sparse_core_info()
_NC, _NS, _L = _info.num_cores, _info.num_subcores, _info.num_lanes
_NW = _NC * _NS
_EPW = N_EDGES // _NW
_UNROLL = 4
_FULL_ITERS = _EPW // (_L * _UNROLL)
_EPW_PAD = (_EPW + _L - 1) // _L * _L


def _matvec_body(x_ref, w_ref, b_ref, o_ref):
    o_ref[...] = jnp.sum(x_ref[...] * w_ref[...], axis=1) + b_ref[0, 0]


def _node_scores(task_repr, wv, db):
    return pl.pallas_call(
        _matvec_body,
        grid=((N_NODES + _ROWS_PER_BLK - 1) // _ROWS_PER_BLK,),
        in_specs=[
            pl.BlockSpec((_ROWS_PER_BLK, D_FEAT), lambda i: (i, 0)),
            pl.BlockSpec((1, D_FEAT), lambda i: (0, 0)),
            pl.BlockSpec((1, 1), lambda i: (0, 0)),
        ],
        out_specs=pl.BlockSpec((_ROWS_PER_BLK,), lambda i: (i,)),
        out_shape=jax.ShapeDtypeStruct((N_NODES,), jnp.float32),
    )(task_repr, wv, db)


_sc_mesh = plsc.VectorSubcoreMesh(core_axis_name="c", subcore_axis_name="s")


@functools.partial(
    pl.kernel,
    out_type=(
        jax.ShapeDtypeStruct((N_EDGES,), jnp.float32),
        jax.ShapeDtypeStruct((N_EDGES,), jnp.float32),
    ),
    mesh=_sc_mesh,
    compiler_params=pltpu.CompilerParams(needs_layout_passes=False),
    scratch_types=[
        pltpu.VMEM((N_NODES,), jnp.float32),
        pltpu.VMEM((_EPW_PAD,), jnp.int32),
        pltpu.VMEM((_EPW_PAD,), jnp.int32),
        pltpu.VMEM((_EPW_PAD,), jnp.float32),
        pltpu.VMEM((_EPW_PAD,), jnp.float32),
        pltpu.SemaphoreType.DMA,
        pltpu.SemaphoreType.DMA,
        pltpu.SemaphoreType.DMA,
    ],
)
def _edge_softmax(d_hbm, edge_hbm, ac_hbm, at_hbm,
                  d_v, row_v, col_v, ac_v, at_v, sem0, sem1, sem2):
    wid = lax.axis_index("s") * _NC + lax.axis_index("c")
    base = wid * _EPW
    cp_d = pltpu.async_copy(d_hbm, d_v, sem0)
    cp_r = pltpu.async_copy(
        edge_hbm.at[pl.ds(base, _EPW)], row_v.at[pl.ds(0, _EPW)], sem1)
    cp_c = pltpu.async_copy(
        edge_hbm.at[pl.ds(N_EDGES + base, _EPW)], col_v.at[pl.ds(0, _EPW)],
        sem2)
    cp_d.wait()
    cp_r.wait()
    cp_c.wait()

    def lanes(off):
        r = row_v[pl.ds(off, _L)]
        c = col_v[pl.ds(off, _L)]
        s = plsc.load_gather(d_v, [r]) + plsc.load_gather(d_v, [c])
        ac = 1.0 / (1.0 + jnp.exp(-s))
        ac_v[pl.ds(off, _L)] = ac
        at_v[pl.ds(off, _L)] = 1.0 - ac

    def body(i, carry):
        for j in range(_UNROLL):
            lanes(i * (_L * _UNROLL) + j * _L)
        return carry

    lax.fori_loop(0, _FULL_ITERS, body, 0)

    lane = lax.iota(jnp.int32, _L)
    for off in range(_FULL_ITERS * _L * _UNROLL, _EPW_PAD, _L):
        n_valid = _EPW - off
        if n_valid >= _L:
            lanes(off)
        else:
            keep = lane < n_valid
            r = jnp.where(keep, row_v[pl.ds(off, _L)], 0)
            c = jnp.where(keep, col_v[pl.ds(off, _L)], 0)
            s = plsc.load_gather(d_v, [r]) + plsc.load_gather(d_v, [c])
            ac = 1.0 / (1.0 + jnp.exp(-s))
            ac_v[pl.ds(off, _L)] = ac
            at_v[pl.ds(off, _L)] = 1.0 - ac

    cp_ac = pltpu.async_copy(
        ac_v.at[pl.ds(0, _EPW)], ac_hbm.at[pl.ds(base, _EPW)], sem0)
    cp_at = pltpu.async_copy(
        at_v.at[pl.ds(0, _EPW)], at_hbm.at[pl.ds(base, _EPW)], sem1)
    cp_ac.wait()
    cp_at.wait()


def kernel(task_repr, task_edge, W, b):
    wv = (W[:, 0] - W[:, 1]).reshape(1, D_FEAT)
    db = ((b[0] - b[1]) * 0.5).reshape(1, 1)
    d = _node_scores(task_repr, wv, db)
    edge = task_edge.reshape(2 * N_EDGES).astype(jnp.int32)
    a_causual, a_trivial = _edge_softmax(d, edge)
    return (a_causual, a_trivial)


# speedup vs baseline: 25.0563x; 1.2165x over previous
import functools

import jax
import jax.numpy as jnp
from jax import lax
from jax.experimental import pallas as pl
from jax.experimental.pallas import tpu as pltpu
from jax.experimental.pallas import tpu_sc as plsc

N_NODES = 10000
N_EDGES = 160000
D_FEAT = 256

_ROWS_PER_BLK = 2048

_info = plsc.get_sparse_core_info()
_NC, _NS, _L = _info.num_cores, _info.num_subcores, _info.num_lanes
_NW = _NC * _NS
_EPW = 4992
_XB = 128
_EPW_PAD = _EPW + _XB


def _matvec_body(x_ref, w_ref, b_ref, o_ref):
    p = jnp.dot(x_ref[...], w_ref[...], preferred_element_type=jnp.float32)
    db = (b_ref[0, 0] - b_ref[0, 1]) * 0.5
    sgn = (1 - 2 * lax.broadcasted_iota(jnp.int32, p.shape, 1)).astype(
        jnp.float32)
    o_ref[...] = jnp.sum(p * sgn, axis=1) + db


def _node_scores(task_repr, W, b2):
    return pl.pallas_call(
        _matvec_body,
        grid=((N_NODES + _ROWS_PER_BLK - 1) // _ROWS_PER_BLK,),
        in_specs=[
            pl.BlockSpec((_ROWS_PER_BLK, D_FEAT), lambda i: (i, 0)),
            pl.BlockSpec((D_FEAT, 2), lambda i: (0, 0)),
            pl.BlockSpec((1, 2), lambda i: (0, 0)),
        ],
        out_specs=pl.BlockSpec((_ROWS_PER_BLK,), lambda i: (i,)),
        out_shape=jax.ShapeDtypeStruct((N_NODES,), jnp.float32),
    )(task_repr, W, b2)


_sc_mesh = plsc.VectorSubcoreMesh(core_axis_name="c", subcore_axis_name="s")


@functools.partial(
    pl.kernel,
    out_type=(
        jax.ShapeDtypeStruct((N_EDGES,), jnp.float32),
        jax.ShapeDtypeStruct((N_EDGES,), jnp.float32),
    ),
    mesh=_sc_mesh,
    compiler_params=pltpu.CompilerParams(needs_layout_passes=False),
    scratch_types=[
        pltpu.VMEM((N_NODES,), jnp.float32),
        pltpu.VMEM((2, _EPW_PAD), jnp.int32),
        pltpu.VMEM((_EPW_PAD,), jnp.float32),
        pltpu.VMEM((_EPW_PAD,), jnp.float32),
        pltpu.SemaphoreType.DMA,
        pltpu.SemaphoreType.DMA,
        pltpu.SemaphoreType.DMA,
        pltpu.SemaphoreType.DMA,
    ],
)
def _edge_softmax(d_hbm, edge_hbm, ac_hbm, at_hbm,
                  d_v, edge_v, ac_v, at_v, sem0, sem1, sem2, sem3):
    wid = lax.axis_index("s") * _NC + lax.axis_index("c")
    base = wid * _EPW
    cp_d = pltpu.async_copy(d_hbm, d_v, sem0)
    cp_e = pltpu.async_copy(
        edge_hbm.at[:, pl.ds(base, _EPW)], edge_v.at[:, pl.ds(0, _EPW)], sem1)
    cp_d.wait()
    cp_e.wait()

    def lanes(off):
        r = edge_v[0, pl.ds(off, _L)]
        c = edge_v[1, pl.ds(off, _L)]
        s = plsc.load_gather(d_v, [r]) + plsc.load_gather(d_v, [c])
        ac = 1.0 / (1.0 + jnp.exp(-s))
        ac_v[pl.ds(off, _L)] = ac
        at_v[pl.ds(off, _L)] = 1.0 - ac

    @plsc.parallel_loop(0, _EPW, step=_L, unroll=8)
    def _loop(off):
        lanes(off)

    cp_ac = pltpu.async_copy(
        ac_v.at[pl.ds(0, _EPW)], ac_hbm.at[pl.ds(base, _EPW)], sem0)
    cp_at = pltpu.async_copy(
        at_v.at[pl.ds(0, _EPW)], at_hbm.at[pl.ds(base, _EPW)], sem1)

    @pl.when(wid < 2)
    def _extra():
        xbase = _NW * _EPW + wid * _XB
        cp_x = pltpu.async_copy(
            edge_hbm.at[:, pl.ds(xbase, _XB)],
            edge_v.at[:, pl.ds(_EPW, _XB)], sem2)
        cp_x.wait()
        for j in range(_XB // _L):
            lanes(_EPW + j * _L)
        cp_xac = pltpu.async_copy(
            ac_v.at[pl.ds(_EPW, _XB)], ac_hbm.at[pl.ds(xbase, _XB)], sem2)
        cp_xat = pltpu.async_copy(
            at_v.at[pl.ds(_EPW, _XB)], at_hbm.at[pl.ds(xbase, _XB)], sem3)
        cp_xac.wait()
        cp_xat.wait()

    cp_ac.wait()
    cp_at.wait()


def kernel(task_repr, task_edge, W, b):
    d = _node_scores(task_repr, W, b.reshape(1, 2))
    a_causual, a_trivial = _edge_softmax(d, task_edge.astype(jnp.int32))
    return (a_causual, a_trivial)


# speedup vs baseline: 27.4382x; 1.0951x over previous
import functools

import jax
import jax.numpy as jnp
from jax import lax
from jax.experimental import pallas as pl
from jax.experimental.pallas import tpu as pltpu
from jax.experimental.pallas import tpu_sc as plsc

N_NODES = 10000
N_EDGES = 160000
D_FEAT = 256

_ROWS_PER_BLK = 2048

_info = plsc.get_sparse_core_info()
_NC, _NS, _L = _info.num_cores, _info.num_subcores, _info.num_lanes
_NW = _NC * _NS
_EPW = 4992
_XB = 128
_EPW_PAD = _EPW + _XB


def _matvec_body(x_ref, w_ref, b_ref, o_ref):
    wv = w_ref[:, 0:1] - w_ref[:, 1:2]
    p = jnp.dot(x_ref[...], wv, preferred_element_type=jnp.float32)
    db = (b_ref[0, 0] - b_ref[0, 1]) * 0.5
    o_ref[...] = p[:, 0] + db


def _node_scores(task_repr, W, b2):
    return pl.pallas_call(
        _matvec_body,
        grid=((N_NODES + _ROWS_PER_BLK - 1) // _ROWS_PER_BLK,),
        in_specs=[
            pl.BlockSpec((_ROWS_PER_BLK, D_FEAT), lambda i: (i, 0)),
            pl.BlockSpec((D_FEAT, 2), lambda i: (0, 0)),
            pl.BlockSpec((1, 2), lambda i: (0, 0)),
        ],
        out_specs=pl.BlockSpec((_ROWS_PER_BLK,), lambda i: (i,)),
        out_shape=jax.ShapeDtypeStruct((N_NODES,), jnp.float32),
    )(task_repr, W, b2)


_sc_mesh = plsc.VectorSubcoreMesh(core_axis_name="c", subcore_axis_name="s")


@functools.partial(
    pl.kernel,
    out_type=(
        jax.ShapeDtypeStruct((N_EDGES,), jnp.float32),
        jax.ShapeDtypeStruct((N_EDGES,), jnp.float32),
    ),
    mesh=_sc_mesh,
    compiler_params=pltpu.CompilerParams(needs_layout_passes=False),
    scratch_types=[
        pltpu.VMEM((N_NODES,), jnp.float32),
        pltpu.VMEM((2, _EPW_PAD), jnp.int32),
        pltpu.VMEM((_EPW_PAD,), jnp.float32),
        pltpu.VMEM((_EPW_PAD,), jnp.float32),
        pltpu.SemaphoreType.DMA,
        pltpu.SemaphoreType.DMA,
        pltpu.SemaphoreType.DMA,
        pltpu.SemaphoreType.DMA,
    ],
)
def _edge_softmax(d_hbm, edge_hbm, ac_hbm, at_hbm,
                  d_v, edge_v, ac_v, at_v, sem0, sem1, sem2, sem3):
    wid = lax.axis_index("s") * _NC + lax.axis_index("c")
    base = wid * _EPW
    cp_d = pltpu.async_copy(d_hbm, d_v, sem0)
    cp_e = pltpu.async_copy(
        edge_hbm.at[:, pl.ds(base, _EPW)], edge_v.at[:, pl.ds(0, _EPW)], sem1)
    cp_d.wait()
    cp_e.wait()

    def lanes(off):
        r = edge_v[0, pl.ds(off, _L)]
        c = edge_v[1, pl.ds(off, _L)]
        s = plsc.load_gather(d_v, [r]) + plsc.load_gather(d_v, [c])
        ac = 1.0 / (1.0 + jnp.exp(-s))
        ac_v[pl.ds(off, _L)] = ac
        at_v[pl.ds(off, _L)] = 1.0 - ac

    @plsc.parallel_loop(0, _EPW, step=_L, unroll=8)
    def _loop(off):
        lanes(off)

    cp_ac = pltpu.async_copy(
        ac_v.at[pl.ds(0, _EPW)], ac_hbm.at[pl.ds(base, _EPW)], sem0)
    cp_at = pltpu.async_copy(
        at_v.at[pl.ds(0, _EPW)], at_hbm.at[pl.ds(base, _EPW)], sem1)

    @pl.when(wid < 2)
    def _extra():
        xbase = _NW * _EPW + wid * _XB
        cp_x = pltpu.async_copy(
            edge_hbm.at[:, pl.ds(xbase, _XB)],
            edge_v.at[:, pl.ds(_EPW, _XB)], sem2)
        cp_x.wait()
        for j in range(_XB // _L):
            lanes(_EPW + j * _L)
        cp_xac = pltpu.async_copy(
            ac_v.at[pl.ds(_EPW, _XB)], ac_hbm.at[pl.ds(xbase, _XB)], sem2)
        cp_xat = pltpu.async_copy(
            at_v.at[pl.ds(_EPW, _XB)], at_hbm.at[pl.ds(xbase, _XB)], sem3)
        cp_xac.wait()
        cp_xat.wait()

    cp_ac.wait()
    cp_at.wait()


def kernel(task_repr, task_edge, W, b):
    d = _node_scores(task_repr, W, b.reshape(1, 2))
    a_causual, a_trivial = _edge_softmax(d, task_edge.astype(jnp.int32))
    return (a_causual, a_trivial)


# speedup vs baseline: 28.7110x; 1.0464x over previous
import functools

import jax
import jax.numpy as jnp
from jax import lax
from jax.experimental import pallas as pl
from jax.experimental.pallas import tpu as pltpu
from jax.experimental.pallas import tpu_sc as plsc

N_NODES = 10000
N_EDGES = 160000
D_FEAT = 256

_ROWS_PER_BLK = 2048

_info = plsc.get_sparse_core_info()
_NC, _NS, _L = _info.num_cores, _info.num_subcores, _info.num_lanes
_NW = _NC * _NS
_EPW = 4992
_XB = 128
_EPW_PAD = _EPW + _XB


def _matvec_body(x_ref, wt_ref, b_ref, o_ref):
    wv = wt_ref[0:1, :] - wt_ref[1:2, :]
    db = (b_ref[0, 0] - b_ref[0, 1]) * 0.5
    o_ref[...] = jnp.sum(x_ref[...] * wv, axis=1) + db


def _node_scores(task_repr, Wt, b2):
    return pl.pallas_call(
        _matvec_body,
        grid=((N_NODES + _ROWS_PER_BLK - 1) // _ROWS_PER_BLK,),
        in_specs=[
            pl.BlockSpec((_ROWS_PER_BLK, D_FEAT), lambda i: (i, 0)),
            pl.BlockSpec((2, D_FEAT), lambda i: (0, 0)),
            pl.BlockSpec((1, 2), lambda i: (0, 0)),
        ],
        out_specs=pl.BlockSpec((_ROWS_PER_BLK,), lambda i: (i,)),
        out_shape=jax.ShapeDtypeStruct((N_NODES,), jnp.float32),
    )(task_repr, Wt, b2)


_sc_mesh = plsc.VectorSubcoreMesh(core_axis_name="c", subcore_axis_name="s")


@functools.partial(
    pl.kernel,
    out_type=(
        jax.ShapeDtypeStruct((N_EDGES,), jnp.float32),
        jax.ShapeDtypeStruct((N_EDGES,), jnp.float32),
    ),
    mesh=_sc_mesh,
    compiler_params=pltpu.CompilerParams(needs_layout_passes=False),
    scratch_types=[
        pltpu.VMEM((N_NODES,), jnp.float32),
        pltpu.VMEM((2, _EPW_PAD), jnp.int32),
        pltpu.VMEM((_EPW_PAD,), jnp.float32),
        pltpu.VMEM((_EPW_PAD,), jnp.float32),
        pltpu.SemaphoreType.DMA,
        pltpu.SemaphoreType.DMA,
        pltpu.SemaphoreType.DMA,
        pltpu.SemaphoreType.DMA,
    ],
)
def _edge_softmax(d_hbm, edge_hbm, ac_hbm, at_hbm,
                  d_v, edge_v, ac_v, at_v, sem0, sem1, sem2, sem3):
    wid = lax.axis_index("s") * _NC + lax.axis_index("c")
    base = wid * _EPW
    cp_d = pltpu.async_copy(d_hbm, d_v, sem0)
    cp_e = pltpu.async_copy(
        edge_hbm.at[:, pl.ds(base, _EPW)], edge_v.at[:, pl.ds(0, _EPW)], sem1)
    cp_d.wait()
    cp_e.wait()

    def lanes(off):
        r = edge_v[0, pl.ds(off, _L)]
        c = edge_v[1, pl.ds(off, _L)]
        s = plsc.load_gather(d_v, [r]) + plsc.load_gather(d_v, [c])
        ac = 1.0 / (1.0 + jnp.exp(-s))
        ac_v[pl.ds(off, _L)] = ac
        at_v[pl.ds(off, _L)] = 1.0 - ac

    @plsc.parallel_loop(0, _EPW, step=_L, unroll=12)
    def _loop(off):
        lanes(off)

    cp_ac = pltpu.async_copy(
        ac_v.at[pl.ds(0, _EPW)], ac_hbm.at[pl.ds(base, _EPW)], sem0)
    cp_at = pltpu.async_copy(
        at_v.at[pl.ds(0, _EPW)], at_hbm.at[pl.ds(base, _EPW)], sem1)

    @pl.when(wid < 2)
    def _extra():
        xbase = _NW * _EPW + wid * _XB
        cp_x = pltpu.async_copy(
            edge_hbm.at[:, pl.ds(xbase, _XB)],
            edge_v.at[:, pl.ds(_EPW, _XB)], sem2)
        cp_x.wait()
        for j in range(_XB // _L):
            lanes(_EPW + j * _L)
        cp_xac = pltpu.async_copy(
            ac_v.at[pl.ds(_EPW, _XB)], ac_hbm.at[pl.ds(xbase, _XB)], sem2)
        cp_xat = pltpu.async_copy(
            at_v.at[pl.ds(_EPW, _XB)], at_hbm.at[pl.ds(xbase, _XB)], sem3)
        cp_xac.wait()
        cp_xat.wait()

    cp_ac.wait()
    cp_at.wait()


def kernel(task_repr, task_edge, W, b):
    d = _node_scores(task_repr, W.T, b.reshape(1, 2))
    a_causual, a_trivial = _edge_softmax(d, task_edge.astype(jnp.int32))
    return (a_causual, a_trivial)
